# baseline (speedup 1.0000x reference)
import functools

import jax
import jax.numpy as jnp
from jax import lax
from jax.experimental import pallas as pl
from jax.experimental.pallas import tpu as pltpu
from jax.experimental.pallas import tpu_sc as plsc

_NC = 2
_NS = 16
_LANES = 16


def _pick_chunk(edges_per_worker: int) -> int:
    for c in range(40, 0, -8):
        if edges_per_worker % c == 0:
            return c
    return 0


@functools.partial(jax.jit, static_argnums=(3,))
def _edge_accumulate(x_src, esrc, edst, n_dst):
    _, k_chunks, chunk = esrc.shape
    d = x_src.shape[1]
    cw = _LANES
    n_acc = -(-n_dst // (_NS * 128)) * (_NS * 128)
    rows_per_tile = n_acc // _NS
    zrows = 16
    zreps = rows_per_tile // zrows
    nbuf = 6
    nidx = 12
    assert k_chunks >= 3 * nidx

    mesh = plsc.VectorSubcoreMesh(core_axis_name="c", subcore_axis_name="s")

    @functools.partial(
        pl.kernel,
        out_type=(
            jax.ShapeDtypeStruct((_NC, n_acc, d), jnp.float32),
            jax.ShapeDtypeStruct((_NC, n_acc, cw), jnp.float32),
        ),
        mesh=mesh,
        scratch_types=[
            pltpu.VMEM((nidx, chunk), jnp.int32),
            pltpu.VMEM((nidx, chunk), jnp.int32),
            [pltpu.VMEM((chunk, d), jnp.float32) for _ in range(nbuf)],
            pltpu.VMEM((chunk, cw), jnp.float32),
            pltpu.VMEM((zrows, d), jnp.float32),
            pltpu.VMEM((zrows, cw), jnp.float32),
            pltpu.VMEM_SHARED((n_acc, d), jnp.float32),
            pltpu.VMEM_SHARED((n_acc, cw), jnp.float32),
            [pltpu.SemaphoreType.DMA for _ in range(nidx)],
            [pltpu.SemaphoreType.DMA for _ in range(nbuf)],
            [pltpu.SemaphoreType.DMA for _ in range(nbuf)],
            [pltpu.SemaphoreType.DMA for _ in range(nbuf)],
        ],
        compiler_params=pltpu.CompilerParams(use_tc_tiling_on_sc=False),
    )
    def body(x_hbm, esrc_hbm, edst_hbm, outf_hbm, outc_hbm,
             sidx, didx, bufs, obuf, zbuf, zbufc, accf, accc,
             isems, gsems, ssems, csems):
        cid = lax.axis_index("c")
        sid = lax.axis_index("s")
        wid = cid * _NS + sid
        esrc_w = esrc_hbm.at[wid]
        edst_w = edst_hbm.at[wid]

        def idx_start(j, s):
            pltpu.async_copy(esrc_w.at[j], sidx.at[s], isems[s])
            pltpu.async_copy(edst_w.at[j], didx.at[s], isems[s])

        def idx_wait(j, s):
            pltpu.make_async_copy(esrc_w.at[j], sidx.at[s], isems[s]).wait()
            pltpu.make_async_copy(edst_w.at[j], didx.at[s], isems[s]).wait()

        def gather_start(s, b):
            pltpu.async_copy(x_hbm.at[sidx.at[s]], bufs[b], gsems[b])

        def gather_wait(s, b):
            pltpu.make_async_copy(x_hbm.at[sidx.at[s]], bufs[b], gsems[b]).wait()

        def scatter_start(s, b):
            pltpu.async_copy(bufs[b], accf.at[didx.at[s]], ssems[b], add=True)
            pltpu.async_copy(obuf, accc.at[didx.at[s]], csems[b], add=True)

        def scatter_wait(s, b):
            pltpu.make_async_copy(bufs[b], accf.at[didx.at[s]], ssems[b]).wait()
            pltpu.make_async_copy(obuf, accc.at[didx.at[s]], csems[b]).wait()

        for j in range(4):
            idx_start(j, j)

        zvec = jnp.zeros((_LANES,), jnp.float32)
        ovec = jnp.ones((_LANES,), jnp.float32)

        def fill_ones(r, _):
            obuf[r, pl.ds(0, cw)] = ovec
            return 0

        lax.fori_loop(0, chunk, fill_ones, 0)

        def zero_row(r, _):
            for c in range(d // _LANES):
                zbuf[r, pl.ds(c * _LANES, _LANES)] = zvec
            zbufc[r, pl.ds(0, cw)] = zvec
            return 0

        lax.fori_loop(0, zrows, zero_row, 0)
        row0 = sid * rows_per_tile
        for r in range(zreps):
            pltpu.sync_copy(zbuf, accf.at[pl.ds(row0 + r * zrows, zrows)])
            pltpu.sync_copy(zbufc, accc.at[pl.ds(row0 + r * zrows, zrows)])
        plsc.subcore_barrier()

        def step(j, m, do_g2, do_swait, do_i4):
            m2, m4 = (m + 2) % nidx, (m + 4) % nidx
            b, b2 = m % nbuf, (m + 2) % nbuf
            if do_g2:
                idx_wait(j + 2, m2)
                if do_swait:
                    scatter_wait(m2, b2)
                gather_start(m2, b2)
            gather_wait(m, b)
            scatter_start(m, b)
            if do_i4:
                idx_start(j + 4, m4)

        idx_wait(0, 0)
        gather_start(0, 0)
        idx_wait(1, 1)
        gather_start(1, 1)
        for j in range(nidx):
            step(j, j, True, j >= nbuf - 2, True)

        groups = (k_chunks - 4 - nidx) // nidx

        def group(p, _):
            j0 = p * nidx
            for m in range(nidx):
                step(j0 + m, m, True, True, True)
            return 0

        lax.fori_loop(1, groups + 1, group, 0)

        for j in range((groups + 1) * nidx, k_chunks):
            do_g2 = j + 2 < k_chunks
            step(j, j % nidx, do_g2, do_g2, j + 4 < k_chunks)

        for j in range(k_chunks - nbuf, k_chunks):
            scatter_wait(j % nidx, j % nbuf)

        plsc.subcore_barrier()
        pltpu.sync_copy(
            accf.at[pl.ds(row0, rows_per_tile)],
            outf_hbm.at[cid].at[pl.ds(row0, rows_per_tile)],
        )
        pltpu.sync_copy(
            accc.at[pl.ds(row0, rows_per_tile)],
            outc_hbm.at[cid].at[pl.ds(row0, rows_per_tile)],
        )

    return body(x_src, esrc, edst)


def _dense_combine(s2, c2, x_dst, W1, b1, W2, b2, Wn, bn, Ws, bs):
    n, d = x_dst.shape
    cw = c2.shape[-1]
    rows = n
    for r in (1000, 500, 200, 100, 50, 25, 8):
        if n % r == 0 and r % 8 == 0:
            rows = r
            break
    grid = n // rows

    def body(s2_ref, c2_ref, x_ref, w1_ref, b1_ref, w2_ref, b2_ref,
             wn_ref, bn_ref, ws_ref, bs_ref, o_ref):
        feats = s2_ref[0] + s2_ref[1]
        cnt = c2_ref[0, :, :1] + c2_ref[1, :, :1]
        x = x_ref[...]
        h = jnp.maximum(
            jnp.dot(x, w1_ref[...], preferred_element_type=jnp.float32)
            + b1_ref[...], 0.0)
        pred = (jnp.dot(h, w2_ref[...], preferred_element_type=jnp.float32)
                + b2_ref[...])
        inv = 1.0 / jnp.maximum(cnt, 1.0)
        mask = (cnt > 0.0).astype(jnp.float32)
        agg = feats * inv - pred * mask
        o_ref[...] = (
            jnp.dot(agg, wn_ref[...], preferred_element_type=jnp.float32)
            + jnp.dot(x, ws_ref[...], preferred_element_type=jnp.float32)
            + bn_ref[...] + bs_ref[...])

    wspec = pl.BlockSpec((d, d), lambda i: (0, 0))
    bspec = pl.BlockSpec((1, d), lambda i: (0, 0))
    return pl.pallas_call(
        body,
        grid=(grid,),
        in_specs=[
            pl.BlockSpec((_NC, rows, d), lambda i: (0, i, 0)),
            pl.BlockSpec((_NC, rows, cw), lambda i: (0, i, 0)),
            pl.BlockSpec((rows, d), lambda i: (i, 0)),
            wspec, bspec, wspec, bspec, wspec, bspec, wspec, bspec,
        ],
        out_specs=pl.BlockSpec((rows, d), lambda i: (i, 0)),
        out_shape=jax.ShapeDtypeStruct((n, d), jnp.float32),
    )(s2, c2, x_dst, W1, b1.reshape(1, d), W2, b2.reshape(1, d),
      Wn, bn.reshape(1, d), Ws, bs.reshape(1, d))


def kernel(x_src, x_dst, edge_src, edge_dst, num_dst,
           W1, b1, W2, b2, Wn, bn, Ws, bs):
    n_src, d = x_src.shape
    n_dst = x_dst.shape[0]
    e = edge_src.shape[0]
    n_worker = _NC * _NS

    assert d % _LANES == 0 and n_dst % _NS == 0 and e % n_worker == 0
    edges_per_worker = e // n_worker
    chunk = _pick_chunk(edges_per_worker)
    k_chunks = edges_per_worker // chunk
    assert chunk > 0 and k_chunks >= 24

    esrc = edge_src.astype(jnp.int32).reshape(n_worker, k_chunks, chunk)
    edst = edge_dst.astype(jnp.int32).reshape(n_worker, k_chunks, chunk)

    s2, c2 = _edge_accumulate(x_src.astype(jnp.float32), esrc, edst, n_dst)
    return _dense_combine(s2, c2, x_dst.astype(jnp.float32),
                          W1, b1, W2, b2, Wn, bn, Ws, bs)


# speedup vs baseline: 1.0037x; 1.0037x over previous
import functools

import jax
import jax.numpy as jnp
from jax import lax
from jax.experimental import pallas as pl
from jax.experimental.pallas import tpu as pltpu
from jax.experimental.pallas import tpu_sc as plsc

_NC = 2
_NS = 16
_LANES = 16


def _pick_chunk(edges_per_worker: int) -> int:
    for c in range(40, 0, -8):
        if edges_per_worker % c == 0:
            return c
    return 0


@functools.partial(jax.jit, static_argnums=(3,))
def _edge_accumulate(x_src, esrc, edst, n_dst):
    _, k_chunks, chunk = esrc.shape
    d = x_src.shape[1]
    cw = _LANES
    n_acc = -(-n_dst // (_NS * 128)) * (_NS * 128)
    rows_per_tile = n_acc // _NS
    zrows = 16
    zreps = rows_per_tile // zrows
    nbuf = 4
    nidx = 8
    assert k_chunks >= 3 * nidx

    mesh = plsc.VectorSubcoreMesh(core_axis_name="c", subcore_axis_name="s")

    @functools.partial(
        pl.kernel,
        out_type=(
            jax.ShapeDtypeStruct((_NC, n_acc, d), jnp.float32),
            jax.ShapeDtypeStruct((_NC, n_acc, cw), jnp.float32),
        ),
        mesh=mesh,
        scratch_types=[
            pltpu.VMEM((nidx, chunk), jnp.int32),
            pltpu.VMEM((nidx, chunk), jnp.int32),
            [pltpu.VMEM((chunk, d), jnp.float32) for _ in range(nbuf)],
            pltpu.VMEM((chunk, cw), jnp.float32),
            pltpu.VMEM((zrows, d), jnp.float32),
            pltpu.VMEM((zrows, cw), jnp.float32),
            pltpu.VMEM_SHARED((n_acc, d), jnp.float32),
            pltpu.VMEM_SHARED((n_acc, cw), jnp.float32),
            [pltpu.SemaphoreType.DMA for _ in range(nidx)],
            [pltpu.SemaphoreType.DMA for _ in range(nbuf)],
            [pltpu.SemaphoreType.DMA for _ in range(nbuf)],
            [pltpu.SemaphoreType.DMA for _ in range(nbuf)],
        ],
        compiler_params=pltpu.CompilerParams(use_tc_tiling_on_sc=False),
    )
    def body(x_hbm, esrc_hbm, edst_hbm, outf_hbm, outc_hbm,
             sidx, didx, bufs, obuf, zbuf, zbufc, accf, accc,
             isems, gsems, ssems, csems):
        cid = lax.axis_index("c")
        sid = lax.axis_index("s")
        wid = cid * _NS + sid
        esrc_w = esrc_hbm.at[wid]
        edst_w = edst_hbm.at[wid]

        def idx_start(j, s):
            pltpu.async_copy(esrc_w.at[j], sidx.at[s], isems[s])
            pltpu.async_copy(edst_w.at[j], didx.at[s], isems[s])

        def idx_wait(j, s):
            pltpu.make_async_copy(esrc_w.at[j], sidx.at[s], isems[s]).wait()
            pltpu.make_async_copy(edst_w.at[j], didx.at[s], isems[s]).wait()

        def gather_start(s, b):
            pltpu.async_copy(x_hbm.at[sidx.at[s]], bufs[b], gsems[b])

        def gather_wait(s, b):
            pltpu.make_async_copy(x_hbm.at[sidx.at[s]], bufs[b], gsems[b]).wait()

        def scatter_start(s, b):
            pltpu.async_copy(bufs[b], accf.at[didx.at[s]], ssems[b], add=True)
            pltpu.async_copy(obuf, accc.at[didx.at[s]], csems[b], add=True)

        def scatter_wait(s, b):
            pltpu.make_async_copy(bufs[b], accf.at[didx.at[s]], ssems[b]).wait()
            pltpu.make_async_copy(obuf, accc.at[didx.at[s]], csems[b]).wait()

        for j in range(nbuf):
            idx_start(j, j)

        zvec = jnp.zeros((_LANES,), jnp.float32)
        ovec = jnp.ones((_LANES,), jnp.float32)

        def fill_ones(r, _):
            obuf[r, pl.ds(0, cw)] = ovec
            return 0

        lax.fori_loop(0, chunk, fill_ones, 0)

        def zero_row(r, _):
            for c in range(d // _LANES):
                zbuf[r, pl.ds(c * _LANES, _LANES)] = zvec
            zbufc[r, pl.ds(0, cw)] = zvec
            return 0

        lax.fori_loop(0, zrows, zero_row, 0)
        row0 = sid * rows_per_tile
        for r in range(zreps):
            pltpu.sync_copy(zbuf, accf.at[pl.ds(row0 + r * zrows, zrows)])
            pltpu.sync_copy(zbufc, accc.at[pl.ds(row0 + r * zrows, zrows)])
        plsc.subcore_barrier()

        def step(j, m, do_g2, do_swait, do_i4):
            m2, m4 = (m + 2) % nidx, (m + 4) % nidx
            b, b2 = m % nbuf, (m + 2) % nbuf
            if do_g2:
                idx_wait(j + 2, m2)
                if do_swait:
                    scatter_wait(m2, b2)
                gather_start(m2, b2)
            gather_wait(m, b)
            scatter_start(m, b)
            if do_i4:
                idx_start(j + 4, m4)

        idx_wait(0, 0)
        gather_start(0, 0)
        idx_wait(1, 1)
        gather_start(1, 1)
        for j in range(nidx):
            step(j, j, True, j >= 2, True)

        groups = (k_chunks - 12) // nidx

        def group(p, _):
            j0 = p * nidx
            for m in range(nidx):
                step(j0 + m, m, True, True, True)
            return 0

        lax.fori_loop(1, groups + 1, group, 0)

        for j in range((groups + 1) * nidx, k_chunks):
            do_g2 = j + 2 < k_chunks
            step(j, j % nidx, do_g2, do_g2, j + 4 < k_chunks)

        for j in range(k_chunks - nbuf, k_chunks):
            scatter_wait(j % nidx, j % nbuf)

        plsc.subcore_barrier()
        pltpu.sync_copy(
            accf.at[pl.ds(row0, rows_per_tile)],
            outf_hbm.at[cid].at[pl.ds(row0, rows_per_tile)],
        )
        pltpu.sync_copy(
            accc.at[pl.ds(row0, rows_per_tile)],
            outc_hbm.at[cid].at[pl.ds(row0, rows_per_tile)],
        )

    return body(x_src, esrc, edst)


def _dense_combine(s2, c2, x_dst, W1, b1, W2, b2, Wn, bn, Ws, bs):
    n, d = x_dst.shape
    cw = c2.shape[-1]
    rows = n
    for r in (1000, 500, 200, 100, 50, 25, 8):
        if n % r == 0 and r % 8 == 0:
            rows = r
            break
    grid = n // rows

    def body(s2_ref, c2_ref, x_ref, w1_ref, b1_ref, w2_ref, b2_ref,
             wn_ref, bn_ref, ws_ref, bs_ref, o_ref):
        feats = s2_ref[0] + s2_ref[1]
        cnt = c2_ref[0, :, :1] + c2_ref[1, :, :1]
        x = x_ref[...]
        h = jnp.maximum(
            jnp.dot(x, w1_ref[...], preferred_element_type=jnp.float32)
            + b1_ref[...], 0.0)
        pred = (jnp.dot(h, w2_ref[...], preferred_element_type=jnp.float32)
                + b2_ref[...])
        inv = 1.0 / jnp.maximum(cnt, 1.0)
        mask = (cnt > 0.0).astype(jnp.float32)
        agg = feats * inv - pred * mask
        o_ref[...] = (
            jnp.dot(agg, wn_ref[...], preferred_element_type=jnp.float32)
            + jnp.dot(x, ws_ref[...], preferred_element_type=jnp.float32)
            + bn_ref[...] + bs_ref[...])

    wspec = pl.BlockSpec((d, d), lambda i: (0, 0))
    bspec = pl.BlockSpec((1, d), lambda i: (0, 0))
    return pl.pallas_call(
        body,
        grid=(grid,),
        in_specs=[
            pl.BlockSpec((_NC, rows, d), lambda i: (0, i, 0)),
            pl.BlockSpec((_NC, rows, cw), lambda i: (0, i, 0)),
            pl.BlockSpec((rows, d), lambda i: (i, 0)),
            wspec, bspec, wspec, bspec, wspec, bspec, wspec, bspec,
        ],
        out_specs=pl.BlockSpec((rows, d), lambda i: (i, 0)),
        out_shape=jax.ShapeDtypeStruct((n, d), jnp.float32),
    )(s2, c2, x_dst, W1, b1.reshape(1, d), W2, b2.reshape(1, d),
      Wn, bn.reshape(1, d), Ws, bs.reshape(1, d))


def kernel(x_src, x_dst, edge_src, edge_dst, num_dst,
           W1, b1, W2, b2, Wn, bn, Ws, bs):
    n_src, d = x_src.shape
    n_dst = x_dst.shape[0]
    e = edge_src.shape[0]
    n_worker = _NC * _NS

    assert d % _LANES == 0 and n_dst % _NS == 0 and e % n_worker == 0
    edges_per_worker = e // n_worker
    chunk = _pick_chunk(edges_per_worker)
    k_chunks = edges_per_worker // chunk
    assert chunk > 0 and k_chunks >= 24

    esrc = edge_src.astype(jnp.int32).reshape(n_worker, k_chunks, chunk)
    edst = edge_dst.astype(jnp.int32).reshape(n_worker, k_chunks, chunk)

    s2, c2 = _edge_accumulate(x_src.astype(jnp.float32), esrc, edst, n_dst)
    return _dense_combine(s2, c2, x_dst.astype(jnp.float32),
                          W1, b1, W2, b2, Wn, bn, Ws, bs)
